# initial kernel scaffold (unmeasured)
import jax
import jax.numpy as jnp
from jax import lax
from jax.experimental import pallas as pl
from jax.experimental.pallas import tpu as pltpu

N_DEV = 16
BLK = 256


def kernel(x, w_mat, scale_x, scale_w):
    m_total, k_blk = x.shape
    k_total, n_out = w_mat.shape

    def body(x_ref, w_ref, sx_ref, sw_ref, out_ref, xr_ref, send_sems, recv_sems):
        me = lax.axis_index("i")

        sends = []
        for d in range(1, N_DEV):
            t = (me + d) % N_DEV
            rdma = pltpu.make_async_remote_copy(
                src_ref=x_ref.at[pl.ds(t * BLK, BLK), :],
                dst_ref=xr_ref.at[:, pl.ds(me * BLK, BLK)],
                send_sem=send_sems.at[d],
                recv_sem=recv_sems.at[d],
                device_id=(t,),
                device_id_type=pl.DeviceIdType.MESH,
            )
            rdma.start()
            sends.append(rdma)

        xr_ref[:, pl.ds(me * BLK, BLK)] = x_ref[pl.ds(me * BLK, BLK), :]

        for d in range(1, N_DEV):
            s = (me - d) % N_DEV
            recv = pltpu.make_async_remote_copy(
                src_ref=x_ref.at[pl.ds(s * BLK, BLK), :],
                dst_ref=xr_ref.at[:, pl.ds(s * BLK, BLK)],
                send_sem=send_sems.at[d],
                recv_sem=recv_sems.at[d],
                device_id=((me + d) % N_DEV,),
                device_id_type=pl.DeviceIdType.MESH,
            )
            recv.wait_recv()

        acc = jax.lax.dot_general(
            xr_ref[:, :], w_ref[:, :],
            dimension_numbers=(((1,), (0,)), ((), ())),
            preferred_element_type=jnp.float32,
        )
        out_ref[:, :] = acc * (sx_ref[0] * sw_ref[0])

        for rdma in sends:
            rdma.wait_send()

    return pl.pallas_call(
        body,
        out_shape=jax.ShapeDtypeStruct((BLK, n_out), jnp.float32),
        in_specs=[
            pl.BlockSpec(memory_space=pltpu.VMEM),
            pl.BlockSpec(memory_space=pltpu.VMEM),
            pl.BlockSpec(memory_space=pltpu.SMEM),
            pl.BlockSpec(memory_space=pltpu.SMEM),
        ],
        out_specs=pl.BlockSpec(memory_space=pltpu.VMEM),
        scratch_shapes=[
            pltpu.VMEM((BLK, k_total), x.dtype),
            pltpu.SemaphoreType.DMA((N_DEV,)),
            pltpu.SemaphoreType.DMA((N_DEV,)),
        ],
        compiler_params=pltpu.CompilerParams(collective_id=0),
    )(x, w_mat, scale_x, scale_w)


# baseline (device time: 64588 ns/iter reference)
import jax
import jax.numpy as jnp
from jax import lax
from jax.experimental import pallas as pl
from jax.experimental.pallas import tpu as pltpu

N_DEV = 16
BLK = 256
NC = 512
N_CHUNKS = 8192 // NC


def kernel(x, w_mat, scale_x, scale_w):
    m_total, k_blk = x.shape
    k_total, n_out = w_mat.shape

    def body(x_ref, w_ref, sx_ref, sw_ref, out_ref,
             x8_ref, xr_ref, wbuf_ref, w8_ref, send_sems, recv_sems, wdma_sems):
        me = lax.axis_index("i")

        def wdma(c, slot):
            return pltpu.make_async_copy(
                w_ref.at[:, pl.ds(c * NC, NC)],
                wbuf_ref.at[slot],
                wdma_sems.at[slot],
            )

        wdma(0, 0).start()
        wdma(1, 1).start()

        x8_ref[...] = x_ref[...].astype(jnp.float8_e4m3fn)

        sends = []
        for d in range(1, N_DEV):
            t = (me + d) % N_DEV
            rdma = pltpu.make_async_remote_copy(
                src_ref=x8_ref.at[pl.ds(t * BLK, BLK), :],
                dst_ref=xr_ref.at[:, pl.ds(me * BLK, BLK)],
                send_sem=send_sems.at[d],
                recv_sem=recv_sems.at[d],
                device_id=(t,),
                device_id_type=pl.DeviceIdType.MESH,
            )
            rdma.start()
            sends.append(rdma)

        xr_ref[:, pl.ds(me * BLK, BLK)] = x8_ref[pl.ds(me * BLK, BLK), :]

        for d in range(1, N_DEV):
            s = (me - d) % N_DEV
            recv = pltpu.make_async_remote_copy(
                src_ref=x8_ref.at[pl.ds(s * BLK, BLK), :],
                dst_ref=xr_ref.at[:, pl.ds(s * BLK, BLK)],
                send_sem=send_sems.at[d],
                recv_sem=recv_sems.at[d],
                device_id=((me + d) % N_DEV,),
                device_id_type=pl.DeviceIdType.MESH,
            )
            recv.wait_recv()

        scale = sx_ref[0] * sw_ref[0]

        for c in range(N_CHUNKS):
            slot = c % 2
            wdma(c, slot).wait()
            w8_ref[...] = wbuf_ref[slot].astype(jnp.float8_e5m2)
            if c + 2 < N_CHUNKS:
                wdma(c + 2, slot).start()
            acc = lax.dot_general(
                xr_ref[...], w8_ref[...],
                dimension_numbers=(((1,), (0,)), ((), ())),
                preferred_element_type=jnp.float32,
            )
            out_ref[:, pl.ds(c * NC, NC)] = acc * scale

        for rdma in sends:
            rdma.wait_send()

    return pl.pallas_call(
        body,
        out_shape=jax.ShapeDtypeStruct((BLK, n_out), jnp.float32),
        in_specs=[
            pl.BlockSpec(memory_space=pltpu.VMEM),
            pl.BlockSpec(memory_space=pltpu.MemorySpace.HBM),
            pl.BlockSpec(memory_space=pltpu.SMEM),
            pl.BlockSpec(memory_space=pltpu.SMEM),
        ],
        out_specs=pl.BlockSpec(memory_space=pltpu.VMEM),
        scratch_shapes=[
            pltpu.VMEM((m_total, k_blk), jnp.float8_e4m3fn),
            pltpu.VMEM((BLK, k_total), jnp.float8_e4m3fn),
            pltpu.VMEM((2, k_total, NC), jnp.float32),
            pltpu.VMEM((k_total, NC), jnp.float8_e5m2),
            pltpu.SemaphoreType.DMA((N_DEV,)),
            pltpu.SemaphoreType.DMA((N_DEV,)),
            pltpu.SemaphoreType.DMA((2,)),
        ],
    )(x, w_mat, scale_x, scale_w)


# device time: 63292 ns/iter; 1.0205x vs baseline; 1.0205x over previous
import jax
import jax.numpy as jnp
from jax import lax
from jax.experimental import pallas as pl
from jax.experimental.pallas import tpu as pltpu

N_DEV = 16
BLK = 256
NC = 256
N_CHUNKS = 8192 // NC
DEPTH = 4
RS = 2


def kernel(x, w_mat, scale_x, scale_w):
    m_total, k_blk = x.shape
    k_total, n_out = w_mat.shape

    def body(x_ref, w_ref, sx_ref, sw_ref, out_ref,
             x8_ref, xr_ref, wbuf_ref, w8_ref, send_sems, recv_sems, wdma_sems):
        me = lax.axis_index("i")

        kh = k_total // RS

        def wdma(c, slot):
            return [
                pltpu.make_async_copy(
                    w_ref.at[pl.ds(r * kh, kh), pl.ds(c * NC, NC)],
                    wbuf_ref.at[slot, pl.ds(r * kh, kh)],
                    wdma_sems.at[slot, r],
                )
                for r in range(RS)
            ]

        for c in range(DEPTH):
            for cp in wdma(c, c):
                cp.start()

        x8_ref[...] = x_ref[...].astype(jnp.float8_e4m3fn)

        sends = []
        for d in range(1, N_DEV):
            t = (me + d) % N_DEV
            rdma = pltpu.make_async_remote_copy(
                src_ref=x8_ref.at[pl.ds(t * BLK, BLK), :],
                dst_ref=xr_ref.at[:, pl.ds(me * BLK, BLK)],
                send_sem=send_sems.at[d],
                recv_sem=recv_sems.at[d],
                device_id=(t,),
                device_id_type=pl.DeviceIdType.MESH,
            )
            rdma.start()
            sends.append(rdma)

        xr_ref[:, pl.ds(me * BLK, BLK)] = x8_ref[pl.ds(me * BLK, BLK), :]

        for d in range(1, N_DEV):
            s = (me - d) % N_DEV
            recv = pltpu.make_async_remote_copy(
                src_ref=x8_ref.at[pl.ds(s * BLK, BLK), :],
                dst_ref=xr_ref.at[:, pl.ds(s * BLK, BLK)],
                send_sem=send_sems.at[d],
                recv_sem=recv_sems.at[d],
                device_id=((me + d) % N_DEV,),
                device_id_type=pl.DeviceIdType.MESH,
            )
            recv.wait_recv()

        scale = sx_ref[0] * sw_ref[0]

        for c in range(N_CHUNKS):
            slot = c % DEPTH
            for cp in wdma(c, slot):
                cp.wait()
            w8_ref[...] = wbuf_ref[slot].astype(jnp.float8_e5m2)
            if c + DEPTH < N_CHUNKS:
                for cp in wdma(c + DEPTH, slot):
                    cp.start()
            acc = lax.dot_general(
                xr_ref[...], w8_ref[...],
                dimension_numbers=(((1,), (0,)), ((), ())),
                preferred_element_type=jnp.float32,
            )
            out_ref[:, pl.ds(c * NC, NC)] = acc * scale

        for rdma in sends:
            rdma.wait_send()

    return pl.pallas_call(
        body,
        out_shape=jax.ShapeDtypeStruct((BLK, n_out), jnp.float32),
        in_specs=[
            pl.BlockSpec(memory_space=pltpu.VMEM),
            pl.BlockSpec(memory_space=pltpu.MemorySpace.HBM),
            pl.BlockSpec(memory_space=pltpu.SMEM),
            pl.BlockSpec(memory_space=pltpu.SMEM),
        ],
        out_specs=pl.BlockSpec(memory_space=pltpu.VMEM),
        scratch_shapes=[
            pltpu.VMEM((m_total, k_blk), jnp.float8_e4m3fn),
            pltpu.VMEM((BLK, k_total), jnp.float8_e4m3fn),
            pltpu.VMEM((DEPTH, k_total, NC), jnp.float32),
            pltpu.VMEM((k_total, NC), jnp.float8_e5m2),
            pltpu.SemaphoreType.DMA((N_DEV,)),
            pltpu.SemaphoreType.DMA((N_DEV,)),
            pltpu.SemaphoreType.DMA((DEPTH, RS)),
        ],
    )(x, w_mat, scale_x, scale_w)
